# baseline (speedup 1.0000x reference)
import functools

import jax
import jax.numpy as jnp
from jax import lax
from jax.experimental import pallas as pl
from jax.experimental.pallas import tpu as pltpu
from jax.experimental.pallas import tpu_sc as plsc

N_EXP, N_SKL, H, E, B = 50000, 10000, 128, 300000, 100000
NC, NSUB, L = 2, 16, 16
NW = NC * NSUB
C = 128
CH = (-(-E // (NW * C)) + 7) // 8 * 8
HCH = CH // 2
E_PAD = NW * CH * C
NS_PAD = 10240
QB = 10752
NB = 5
NE_PAD = NB * QB
ACC_E = QB + C
BCH = 25
B_PAD = NW * BCH * C
RB = 400
RBC = 2048


def _mesh():
    return plsc.VectorSubcoreMesh(core_axis_name="c", subcore_axis_name="s",
                                  num_cores=NC, num_subcores=NSUB)


def _fill(ref, rows, wchunks, value):
    def body(t, carry):
        i = t // wchunks
        k = t % wchunks
        ref[i, pl.ds(k * L, L)] = jnp.full((L,), value, jnp.float32)
        return carry
    lax.fori_loop(0, rows * wchunks, body, None)


def _drain(dummy_hbm, ref, sem):
    pltpu.make_async_copy(dummy_hbm, ref, sem).wait()


def _ring_chunks(m, gather, table, srcb, dstb, r0, r1, acc, sg, ss, dummy):
    rbufs = (r0, r1)
    if gather:
        @pl.when(m > 0)
        def _():
            pltpu.async_copy(table.at[srcb.at[0]], r0, sg)

    def pair(g, carry):
        for b in range(2):
            j = g * 2 + b
            rcur = rbufs[b]
            rnxt = rbufs[1 - b]

            @pl.when(j < m)
            def _():
                if gather:
                    @pl.when(j + 1 < m)
                    def _():
                        @pl.when(j >= 1)
                        def _():
                            _drain(dummy, rnxt, ss)
                        pltpu.async_copy(table.at[srcb.at[j + 1]], rnxt, sg)
                    _drain(dummy, rcur, sg)
                else:
                    @pl.when(j >= 1)
                    def _():
                        _drain(dummy, rcur, ss)
                pltpu.async_copy(rcur, acc.at[dstb.at[j]], ss, add=True)
        return carry
    lax.fori_loop(0, (m + 1) // 2, pair, None)
    if gather:
        @pl.when(m >= 2)
        def _():
            _drain(dummy, r0, ss)

    @pl.when(m >= 1)
    def _():
        _drain(dummy, r1, ss)


def _zero_acc(rows_src, acc, base, nfull, rem, sem):
    cps = []
    for k in range(nfull):
        cps.append(pltpu.async_copy(rows_src,
                                    acc.at[pl.ds(base + k * C, C)], sem))
    if rem:
        cps.append(pltpu.async_copy(rows_src.at[pl.ds(0, rem)],
                                    acc.at[pl.ds(base + nfull * C, rem)], sem))
    for cp in cps:
        cp.wait()


def _copy_out(acc, out_ref, r0, r1, src_base, dst_base, nfull, rem, sw):
    rbufs = (r0, r1)
    sizes = [C] * nfull + ([rem] if rem else [])
    nch = len(sizes)
    for k in range(nch):
        rr = sizes[k]
        buf = rbufs[k % 2]
        if k >= 2:
            _drain(out_ref.at[pl.ds(0, sizes[k - 2])],
                   buf.at[pl.ds(0, sizes[k - 2])], sw)
        pltpu.sync_copy(acc.at[pl.ds(src_base + k * C, rr)],
                        buf.at[pl.ds(0, rr)])
        pltpu.async_copy(buf.at[pl.ds(0, rr)],
                         out_ref.at[pl.ds(dst_base + k * C, rr)], sw)
    for k in range(max(0, nch - 2), nch):
        rr = sizes[k]
        _drain(out_ref.at[pl.ds(0, rr)], rbufs[k % 2].at[pl.ds(0, rr)], sw)


@functools.cache
def _make_agg_e2s(gather):
    rpt = NS_PAD // NSUB

    @functools.partial(
        pl.kernel,
        out_type=jax.ShapeDtypeStruct((NC, NS_PAD, H), jnp.float32),
        mesh=_mesh(),
        scratch_types=[
            pltpu.VMEM((HCH, C), jnp.int32),
            pltpu.VMEM((HCH, C), jnp.int32),
            pltpu.VMEM((C, H), jnp.float32),
            pltpu.VMEM((C, H), jnp.float32),
            pltpu.VMEM_SHARED((NS_PAD, H), jnp.float32),
            pltpu.SemaphoreType.DMA,
            pltpu.SemaphoreType.DMA,
        ],
    )
    def agg(table, src_r, dst_r, p_out, srcb, dstb, r0, r1, acc, sg, ss):
        c = lax.axis_index("c")
        s = lax.axis_index("s")
        wid = c * NSUB + s
        dummy = p_out.at[0, pl.ds(0, C)]
        _fill(r0, C, H // L, 0.0)
        _zero_acc(r0, acc, s * rpt, rpt // C, 0, ss)
        if not gather:
            _fill(r0, C, H // L, 1.0)
            _fill(r1, C, H // L, 1.0)
        plsc.subcore_barrier()
        for h in range(2):
            cs = pltpu.async_copy(src_r.at[wid, pl.ds(h * HCH, HCH)], srcb, sg)
            cd = pltpu.async_copy(dst_r.at[wid, pl.ds(h * HCH, HCH)], dstb, sg)
            cs.wait()
            cd.wait()
            _ring_chunks(HCH, gather, table, srcb, dstb, r0, r1, acc,
                         sg, ss, dummy)
        plsc.subcore_barrier()
        _copy_out(acc, p_out.at[c], r0, r1, s * rpt, s * rpt,
                  rpt // C, 0, ss)
    return agg


@functools.cache
def _make_agg_s2e(gather):
    rpt = QB // NSUB
    zpt = ACC_E // NSUB

    @functools.partial(
        pl.kernel,
        out_type=jax.ShapeDtypeStruct((NC, NE_PAD, H), jnp.float32),
        mesh=_mesh(),
        scratch_types=[
            pltpu.VMEM((HCH, C), jnp.int32),
            pltpu.VMEM((HCH, C), jnp.int32),
            pltpu.VMEM((C, H), jnp.float32),
            pltpu.VMEM((C, H), jnp.float32),
            pltpu.VMEM_SHARED((ACC_E, H), jnp.float32),
            pltpu.SemaphoreType.DMA,
            pltpu.SemaphoreType.DMA,
        ],
    )
    def agg(table, src_r, dst_r, p_out, srcb, dstb, r0, r1, acc, sg, ss):
        c = lax.axis_index("c")
        s = lax.axis_index("s")
        wid = c * NSUB + s
        dummy = p_out.at[0, pl.ds(0, C)]
        for b in range(NB):
            _fill(r0, C, H // L, 0.0)
            _zero_acc(r0, acc, s * zpt, zpt // C, zpt % C, ss)
            if not gather:
                _fill(r0, C, H // L, 1.0)
                _fill(r1, C, H // L, 1.0)
            plsc.subcore_barrier()
            for h in range(2):
                if gather:
                    cs = pltpu.async_copy(
                        src_r.at[wid, pl.ds(h * HCH, HCH)], srcb, sg)
                cd = pltpu.async_copy(
                    dst_r.at[wid, pl.ds(h * HCH, HCH)], dstb, sg)
                if gather:
                    cs.wait()
                cd.wait()

                def remap(t, carry):
                    i = t // (C // L)
                    kk = t % (C // L)
                    v = dstb[i, pl.ds(kk * L, L)] - (b * QB)
                    ok = (v >= 0) & (v < QB)
                    dstb[i, pl.ds(kk * L, L)] = jnp.where(ok, v, QB)
                    return carry
                lax.fori_loop(0, HCH * (C // L), remap, None)
                _ring_chunks(HCH, gather, table, srcb, dstb, r0, r1, acc,
                             sg, ss, dummy)
            plsc.subcore_barrier()
            _copy_out(acc, p_out.at[c], r0, r1, s * rpt, b * QB + s * rpt,
                      rpt // C, rpt % C, ss)
            if b < NB - 1:
                plsc.subcore_barrier()
    return agg


@functools.cache
def _make_cls_gather():
    return functools.partial(
        pl.kernel,
        out_type=(
            jax.ShapeDtypeStruct((B_PAD, H), jnp.float32),
            jax.ShapeDtypeStruct((B_PAD, H), jnp.float32),
        ),
        mesh=_mesh(),
        scratch_types=[
            pltpu.VMEM((BCH, C), jnp.int32),
            pltpu.VMEM((BCH, C), jnp.int32),
            pltpu.VMEM((C, H), jnp.float32),
            pltpu.VMEM((C, H), jnp.float32),
            pltpu.VMEM((C, H), jnp.float32),
            pltpu.VMEM((C, H), jnp.float32),
            pltpu.SemaphoreType.DMA,
            pltpu.SemaphoreType.DMA,
        ],
    )(_cls_gather_body)


def _cls_gather_body(s2, e2, ls_r, le_r, gs, ge, lsb, leb, s0, s1, e0, e1,
                     sg, sw):
    c = lax.axis_index("c")
    s = lax.axis_index("s")
    wid = c * NSUB + s
    dummy = gs.at[pl.ds(0, C)]
    cs = pltpu.async_copy(ls_r.at[wid], lsb, sg)
    cd = pltpu.async_copy(le_r.at[wid], leb, sg)
    cs.wait()
    cd.wait()
    sbufs = (s0, s1)
    ebufs = (e0, e1)
    pltpu.async_copy(s2.at[lsb.at[0]], s0, sg)
    pltpu.async_copy(e2.at[leb.at[0]], e0, sg)

    def pair(g, carry):
        for b in range(2):
            j = g * 2 + b
            scur, snxt = sbufs[b], sbufs[1 - b]
            ecur, enxt = ebufs[b], ebufs[1 - b]

            @pl.when(j < BCH)
            def _():
                @pl.when(j + 1 < BCH)
                def _():
                    @pl.when(j >= 1)
                    def _():
                        _drain(dummy, snxt, sw)
                        _drain(dummy, enxt, sw)
                    pltpu.async_copy(s2.at[lsb.at[j + 1]], snxt, sg)
                    pltpu.async_copy(e2.at[leb.at[j + 1]], enxt, sg)
                _drain(dummy, scur, sg)
                _drain(dummy, ecur, sg)
                r0w = wid * (BCH * C) + j * C
                pltpu.async_copy(scur, gs.at[pl.ds(r0w, C)], sw)
                pltpu.async_copy(ecur, ge.at[pl.ds(r0w, C)], sw)
        return carry
    lax.fori_loop(0, (BCH + 1) // 2, pair, None)
    for k in range(2):
        _drain(dummy, sbufs[k], sw)
        _drain(dummy, ebufs[k], sw)


def _make_combine(n, relu):
    def body(p_ref, cnt_ref, x_ref, wl_ref, wr_ref, b_ref, o_ref):
        A = p_ref[0] + p_ref[1]
        S = jnp.dot(A, wl_ref[...], preferred_element_type=jnp.float32)
        cnt = cnt_ref[...]
        cvec = jnp.maximum(cnt[0, :, 0] + cnt[1, :, 0], 1.0)
        O = S / cvec[:, None] + b_ref[...] + jnp.dot(
            x_ref[...], wr_ref[...], preferred_element_type=jnp.float32)
        if relu:
            O = jnp.maximum(O, 0.0)
        o_ref[...] = O

    return pl.pallas_call(
        body,
        grid=(n // RB,),
        in_specs=[
            pl.BlockSpec((NC, RB, H), lambda i: (0, i, 0)),
            pl.BlockSpec((NC, RB, H), lambda i: (0, i, 0)),
            pl.BlockSpec((RB, H), lambda i: (i, 0)),
            pl.BlockSpec((H, H), lambda i: (0, 0)),
            pl.BlockSpec((H, H), lambda i: (0, 0)),
            pl.BlockSpec((1, H), lambda i: (0, 0)),
        ],
        out_specs=pl.BlockSpec((RB, H), lambda i: (i, 0)),
        out_shape=jax.ShapeDtypeStruct((n, H), jnp.float32),
    )


_comb_skl_relu = _make_combine(N_SKL, True)
_comb_skl_lin = _make_combine(N_SKL, False)
_comb_exp_relu = _make_combine(N_EXP, True)
_comb_exp_lin = _make_combine(N_EXP, False)


def _dot_body(gs_ref, ge_ref, o_ref):
    o_ref[...] = jnp.sum(gs_ref[...] * ge_ref[...], axis=1).reshape(RBC // H, H)


_cls_dot = pl.pallas_call(
    _dot_body,
    grid=(B_PAD // RBC,),
    in_specs=[
        pl.BlockSpec((RBC, H), lambda i: (i, 0)),
        pl.BlockSpec((RBC, H), lambda i: (i, 0)),
    ],
    out_specs=pl.BlockSpec((RBC // H, H), lambda i: (i, 0)),
    out_shape=jax.ShapeDtypeStruct((B_PAD // H, H), jnp.float32),
)


def _pad_edges(a, fill):
    pad = jnp.full((E_PAD - E,), fill, jnp.int32)
    return jnp.concatenate([a, pad]).reshape(NW, CH, C)


def kernel(expert_node_id, skill_node_id, e2s_src, e2s_dst, s2e_src, s2e_dst,
           label_expert, label_skill, expert_emb, skill_emb,
           Wl1_e2s, Wr1_e2s, Wl1_s2e, Wr1_s2e, Wl2_e2s, Wr2_e2s, Wl2_s2e,
           Wr2_s2e, bl1_e2s, bl1_s2e, bl2_e2s, bl2_s2e):
    e2s_src_r = _pad_edges(e2s_src, 0)
    e2s_dst_r = _pad_edges(e2s_dst, N_SKL)
    s2e_src_r = _pad_edges(s2e_src, 0)
    s2e_dst_r = _pad_edges(s2e_dst, NB * QB)
    lpad = jnp.zeros((B_PAD - B,), jnp.int32)
    ls_r = jnp.concatenate([label_skill, lpad]).reshape(NW, BCH, C)
    le_r = jnp.concatenate([label_expert, lpad]).reshape(NW, BCH, C)

    agg_e2s = _make_agg_e2s(True)
    agg_s2e = _make_agg_s2e(True)
    cnt_e2s = _make_agg_e2s(False)
    cnt_s2e = _make_agg_s2e(False)

    ones_tab = jnp.ones((8, H), jnp.float32)
    zsrc_r = jnp.zeros((NW, CH, C), jnp.int32)
    cnt_s = cnt_e2s(ones_tab, zsrc_r, e2s_dst_r)
    cnt_e = cnt_s2e(ones_tab, zsrc_r, s2e_dst_r)

    P1s = agg_e2s(expert_emb, e2s_src_r, e2s_dst_r)
    P1e = agg_s2e(skill_emb, s2e_src_r, s2e_dst_r)
    xs1 = _comb_skl_relu(P1s, cnt_s, skill_emb, Wl1_e2s, Wr1_e2s,
                         bl1_e2s.reshape(1, H))
    xe1 = _comb_exp_relu(P1e, cnt_e, expert_emb, Wl1_s2e, Wr1_s2e,
                         bl1_s2e.reshape(1, H))

    P2s = agg_e2s(xe1, e2s_src_r, e2s_dst_r)
    P2e = agg_s2e(xs1, s2e_src_r, s2e_dst_r)
    s2 = _comb_skl_lin(P2s, cnt_s, xs1, Wl2_e2s, Wr2_e2s,
                       bl2_e2s.reshape(1, H))
    e2 = _comb_exp_lin(P2e, cnt_e, xe1, Wl2_s2e, Wr2_s2e,
                       bl2_s2e.reshape(1, H))

    gs, ge = _make_cls_gather()(s2, e2, ls_r, le_r)
    pred = _cls_dot(gs, ge)
    return pred.reshape(B_PAD)[:B]


# speedup vs baseline: 7.6497x; 7.6497x over previous
import functools

import jax
import jax.numpy as jnp
from jax import lax
from jax.experimental import pallas as pl
from jax.experimental.pallas import tpu as pltpu
from jax.experimental.pallas import tpu_sc as plsc

N_EXP, N_SKL, H, E, B = 50000, 10000, 128, 300000, 100000
NC, NSUB, L = 2, 16, 16
NW = NC * NSUB
C = 128
CH = (-(-E // (NW * C)) + 7) // 8 * 8
HCH = CH // 2
E_PAD = NW * CH * C
NS_PAD = 10240
QB = 10752
NB = 5
NE_PAD = NB * QB
ACC_E = QB + C
BCH = 25
B_PAD = NW * BCH * C
RB = 400
RBC = 2048


def _mesh():
    return plsc.VectorSubcoreMesh(core_axis_name="c", subcore_axis_name="s",
                                  num_cores=NC, num_subcores=NSUB)


def _fill(ref, rows, wchunks, value):
    def body(t, carry):
        i = t // wchunks
        k = t % wchunks
        ref[i, pl.ds(k * L, L)] = jnp.full((L,), value, jnp.float32)
        return carry
    lax.fori_loop(0, rows * wchunks, body, None)


def _drain(dummy_hbm, ref, sem):
    pltpu.make_async_copy(dummy_hbm, ref, sem).wait()


def _ring_chunks(m, gather, table, srcb, dstb, r0, r1, acc, sg, ss, dummy):
    rbufs = (r0, r1)
    if gather:
        @pl.when(m > 0)
        def _():
            pltpu.async_copy(table.at[srcb.at[0]], r0, sg)

    def pair(g, carry):
        for b in range(2):
            j = g * 2 + b
            rcur = rbufs[b]
            rnxt = rbufs[1 - b]

            @pl.when(j < m)
            def _():
                if gather:
                    @pl.when(j + 1 < m)
                    def _():
                        @pl.when(j >= 1)
                        def _():
                            _drain(dummy, rnxt, ss)
                        pltpu.async_copy(table.at[srcb.at[j + 1]], rnxt, sg)
                    _drain(dummy, rcur, sg)
                else:
                    @pl.when(j >= 1)
                    def _():
                        _drain(dummy, rcur, ss)
                pltpu.async_copy(rcur, acc.at[dstb.at[j]], ss, add=True)
        return carry
    lax.fori_loop(0, (m + 1) // 2, pair, None)
    if gather:
        @pl.when(m >= 2)
        def _():
            _drain(dummy, r0, ss)

    @pl.when(m >= 1)
    def _():
        _drain(dummy, r1, ss)


def _zero_acc(rows_src, acc, base, nfull, rem, sem):
    cps = []
    for k in range(nfull):
        cps.append(pltpu.async_copy(rows_src,
                                    acc.at[pl.ds(base + k * C, C)], sem))
    if rem:
        cps.append(pltpu.async_copy(rows_src.at[pl.ds(0, rem)],
                                    acc.at[pl.ds(base + nfull * C, rem)], sem))
    for cp in cps:
        cp.wait()


def _copy_out(acc, out_ref, r0, r1, src_base, dst_base, nfull, rem, sw):
    rbufs = (r0, r1)
    sizes = [C] * nfull + ([rem] if rem else [])
    nch = len(sizes)
    for k in range(nch):
        rr = sizes[k]
        buf = rbufs[k % 2]
        if k >= 2:
            _drain(out_ref.at[pl.ds(0, sizes[k - 2])],
                   buf.at[pl.ds(0, sizes[k - 2])], sw)
        pltpu.sync_copy(acc.at[pl.ds(src_base + k * C, rr)],
                        buf.at[pl.ds(0, rr)])
        pltpu.async_copy(buf.at[pl.ds(0, rr)],
                         out_ref.at[pl.ds(dst_base + k * C, rr)], sw)
    for k in range(max(0, nch - 2), nch):
        rr = sizes[k]
        _drain(out_ref.at[pl.ds(0, rr)], rbufs[k % 2].at[pl.ds(0, rr)], sw)


@functools.cache
def _make_agg_e2s(gather):
    rpt = NS_PAD // NSUB

    @functools.partial(
        pl.kernel,
        out_type=jax.ShapeDtypeStruct((NC, NS_PAD, H), jnp.float32),
        mesh=_mesh(),
        scratch_types=[
            pltpu.VMEM((HCH, C), jnp.int32),
            pltpu.VMEM((HCH, C), jnp.int32),
            pltpu.VMEM((C, H), jnp.float32),
            pltpu.VMEM((C, H), jnp.float32),
            pltpu.VMEM_SHARED((NS_PAD, H), jnp.float32),
            pltpu.SemaphoreType.DMA,
            pltpu.SemaphoreType.DMA,
        ],
    )
    def agg(table, src_r, dst_r, p_out, srcb, dstb, r0, r1, acc, sg, ss):
        c = lax.axis_index("c")
        s = lax.axis_index("s")
        wid = c * NSUB + s
        dummy = p_out.at[0, pl.ds(0, C)]
        _fill(r0, C, H // L, 0.0)
        _zero_acc(r0, acc, s * rpt, rpt // C, 0, ss)
        if not gather:
            _fill(r0, C, H // L, 1.0)
            _fill(r1, C, H // L, 1.0)
        plsc.subcore_barrier()
        for h in range(2):
            cs = pltpu.async_copy(src_r.at[wid, pl.ds(h * HCH, HCH)], srcb, sg)
            cd = pltpu.async_copy(dst_r.at[wid, pl.ds(h * HCH, HCH)], dstb, sg)
            cs.wait()
            cd.wait()
            _ring_chunks(HCH, gather, table, srcb, dstb, r0, r1, acc,
                         sg, ss, dummy)
        plsc.subcore_barrier()
        _copy_out(acc, p_out.at[c], r0, r1, s * rpt, s * rpt,
                  rpt // C, 0, ss)
    return agg


@functools.cache
def _make_agg_s2e(gather):
    rpt = QB // NSUB
    zpt = ACC_E // NSUB

    @functools.partial(
        pl.kernel,
        out_type=jax.ShapeDtypeStruct((NC, NE_PAD, H), jnp.float32),
        mesh=_mesh(),
        scratch_types=[
            pltpu.VMEM((HCH, C), jnp.int32),
            pltpu.VMEM((HCH, C), jnp.int32),
            pltpu.VMEM((C, H), jnp.float32),
            pltpu.VMEM((C, H), jnp.float32),
            pltpu.VMEM_SHARED((ACC_E, H), jnp.float32),
            pltpu.SemaphoreType.DMA,
            pltpu.SemaphoreType.DMA,
        ],
    )
    def agg(table, src_r, dst_r, p_out, srcb, dstb, r0, r1, acc, sg, ss):
        c = lax.axis_index("c")
        s = lax.axis_index("s")
        wid = c * NSUB + s
        dummy = p_out.at[0, pl.ds(0, C)]
        for b in range(NB):
            _fill(r0, C, H // L, 0.0)
            _zero_acc(r0, acc, s * zpt, zpt // C, zpt % C, ss)
            if not gather:
                _fill(r0, C, H // L, 1.0)
                _fill(r1, C, H // L, 1.0)
            plsc.subcore_barrier()
            for h in range(2):
                if gather:
                    cs = pltpu.async_copy(
                        src_r.at[wid, pl.ds(h * HCH, HCH)], srcb, sg)
                cd = pltpu.async_copy(
                    dst_r.at[wid, pl.ds(h * HCH, HCH)], dstb, sg)
                if gather:
                    cs.wait()
                cd.wait()

                def remap(t, carry):
                    i = t // (C // L)
                    kk = t % (C // L)
                    v = dstb[i, pl.ds(kk * L, L)] - (b * QB)
                    ok = (v >= 0) & (v < QB)
                    pad = QB + kk * L + lax.iota(jnp.int32, L)
                    dstb[i, pl.ds(kk * L, L)] = jnp.where(ok, v, pad)
                    return carry
                lax.fori_loop(0, HCH * (C // L), remap, None)
                _ring_chunks(HCH, gather, table, srcb, dstb, r0, r1, acc,
                             sg, ss, dummy)
            plsc.subcore_barrier()
            _copy_out(acc, p_out.at[c], r0, r1, s * rpt, b * QB + s * rpt,
                      rpt // C, rpt % C, ss)
            if b < NB - 1:
                plsc.subcore_barrier()
    return agg


@functools.cache
def _make_cls_gather():
    return functools.partial(
        pl.kernel,
        out_type=(
            jax.ShapeDtypeStruct((B_PAD, H), jnp.float32),
            jax.ShapeDtypeStruct((B_PAD, H), jnp.float32),
        ),
        mesh=_mesh(),
        scratch_types=[
            pltpu.VMEM((BCH, C), jnp.int32),
            pltpu.VMEM((BCH, C), jnp.int32),
            pltpu.VMEM((C, H), jnp.float32),
            pltpu.VMEM((C, H), jnp.float32),
            pltpu.VMEM((C, H), jnp.float32),
            pltpu.VMEM((C, H), jnp.float32),
            pltpu.SemaphoreType.DMA,
            pltpu.SemaphoreType.DMA,
        ],
    )(_cls_gather_body)


def _cls_gather_body(s2, e2, ls_r, le_r, gs, ge, lsb, leb, s0, s1, e0, e1,
                     sg, sw):
    c = lax.axis_index("c")
    s = lax.axis_index("s")
    wid = c * NSUB + s
    dummy = gs.at[pl.ds(0, C)]
    cs = pltpu.async_copy(ls_r.at[wid], lsb, sg)
    cd = pltpu.async_copy(le_r.at[wid], leb, sg)
    cs.wait()
    cd.wait()
    sbufs = (s0, s1)
    ebufs = (e0, e1)
    pltpu.async_copy(s2.at[lsb.at[0]], s0, sg)
    pltpu.async_copy(e2.at[leb.at[0]], e0, sg)

    def pair(g, carry):
        for b in range(2):
            j = g * 2 + b
            scur, snxt = sbufs[b], sbufs[1 - b]
            ecur, enxt = ebufs[b], ebufs[1 - b]

            @pl.when(j < BCH)
            def _():
                @pl.when(j + 1 < BCH)
                def _():
                    @pl.when(j >= 1)
                    def _():
                        _drain(dummy, snxt, sw)
                        _drain(dummy, enxt, sw)
                    pltpu.async_copy(s2.at[lsb.at[j + 1]], snxt, sg)
                    pltpu.async_copy(e2.at[leb.at[j + 1]], enxt, sg)
                _drain(dummy, scur, sg)
                _drain(dummy, ecur, sg)
                r0w = wid * (BCH * C) + j * C
                pltpu.async_copy(scur, gs.at[pl.ds(r0w, C)], sw)
                pltpu.async_copy(ecur, ge.at[pl.ds(r0w, C)], sw)
        return carry
    lax.fori_loop(0, (BCH + 1) // 2, pair, None)
    for k in range(2):
        _drain(dummy, sbufs[k], sw)
        _drain(dummy, ebufs[k], sw)


def _make_combine(n, relu):
    def body(p_ref, cnt_ref, x_ref, wl_ref, wr_ref, b_ref, o_ref):
        A = p_ref[0] + p_ref[1]
        S = jnp.dot(A, wl_ref[...], preferred_element_type=jnp.float32)
        cnt = cnt_ref[...]
        cvec = jnp.maximum(cnt[0, :, 0] + cnt[1, :, 0], 1.0)
        O = S / cvec[:, None] + b_ref[...] + jnp.dot(
            x_ref[...], wr_ref[...], preferred_element_type=jnp.float32)
        if relu:
            O = jnp.maximum(O, 0.0)
        o_ref[...] = O

    return pl.pallas_call(
        body,
        grid=(n // RB,),
        in_specs=[
            pl.BlockSpec((NC, RB, H), lambda i: (0, i, 0)),
            pl.BlockSpec((NC, RB, H), lambda i: (0, i, 0)),
            pl.BlockSpec((RB, H), lambda i: (i, 0)),
            pl.BlockSpec((H, H), lambda i: (0, 0)),
            pl.BlockSpec((H, H), lambda i: (0, 0)),
            pl.BlockSpec((1, H), lambda i: (0, 0)),
        ],
        out_specs=pl.BlockSpec((RB, H), lambda i: (i, 0)),
        out_shape=jax.ShapeDtypeStruct((n, H), jnp.float32),
    )


_comb_skl_relu = _make_combine(N_SKL, True)
_comb_skl_lin = _make_combine(N_SKL, False)
_comb_exp_relu = _make_combine(N_EXP, True)
_comb_exp_lin = _make_combine(N_EXP, False)


def _dot_body(gs_ref, ge_ref, o_ref):
    o_ref[...] = jnp.sum(gs_ref[...] * ge_ref[...], axis=1).reshape(RBC // H, H)


_cls_dot = pl.pallas_call(
    _dot_body,
    grid=(B_PAD // RBC,),
    in_specs=[
        pl.BlockSpec((RBC, H), lambda i: (i, 0)),
        pl.BlockSpec((RBC, H), lambda i: (i, 0)),
    ],
    out_specs=pl.BlockSpec((RBC // H, H), lambda i: (i, 0)),
    out_shape=jax.ShapeDtypeStruct((B_PAD // H, H), jnp.float32),
)


def _pad_edges(a, fill):
    pad = fill + jnp.arange(E_PAD - E, dtype=jnp.int32) % C
    return jnp.concatenate([a, pad]).reshape(NW, CH, C)


def kernel(expert_node_id, skill_node_id, e2s_src, e2s_dst, s2e_src, s2e_dst,
           label_expert, label_skill, expert_emb, skill_emb,
           Wl1_e2s, Wr1_e2s, Wl1_s2e, Wr1_s2e, Wl2_e2s, Wr2_e2s, Wl2_s2e,
           Wr2_s2e, bl1_e2s, bl1_s2e, bl2_e2s, bl2_s2e):
    e2s_src_r = _pad_edges(e2s_src, 0)
    e2s_dst_r = _pad_edges(e2s_dst, N_SKL)
    s2e_src_r = _pad_edges(s2e_src, 0)
    s2e_dst_r = _pad_edges(s2e_dst, NB * QB)
    lpad = jnp.zeros((B_PAD - B,), jnp.int32)
    ls_r = jnp.concatenate([label_skill, lpad]).reshape(NW, BCH, C)
    le_r = jnp.concatenate([label_expert, lpad]).reshape(NW, BCH, C)

    agg_e2s = _make_agg_e2s(True)
    agg_s2e = _make_agg_s2e(True)
    cnt_e2s = _make_agg_e2s(False)
    cnt_s2e = _make_agg_s2e(False)

    ones_tab = jnp.ones((8, H), jnp.float32)
    zsrc_r = jnp.zeros((NW, CH, C), jnp.int32)
    cnt_s = cnt_e2s(ones_tab, zsrc_r, e2s_dst_r)
    cnt_e = cnt_s2e(ones_tab, zsrc_r, s2e_dst_r)

    P1s = agg_e2s(expert_emb, e2s_src_r, e2s_dst_r)
    P1e = agg_s2e(skill_emb, s2e_src_r, s2e_dst_r)
    xs1 = _comb_skl_relu(P1s, cnt_s, skill_emb, Wl1_e2s, Wr1_e2s,
                         bl1_e2s.reshape(1, H))
    xe1 = _comb_exp_relu(P1e, cnt_e, expert_emb, Wl1_s2e, Wr1_s2e,
                         bl1_s2e.reshape(1, H))

    P2s = agg_e2s(xe1, e2s_src_r, e2s_dst_r)
    P2e = agg_s2e(xs1, s2e_src_r, s2e_dst_r)
    s2 = _comb_skl_lin(P2s, cnt_s, xs1, Wl2_e2s, Wr2_e2s,
                       bl2_e2s.reshape(1, H))
    e2 = _comb_exp_lin(P2e, cnt_e, xe1, Wl2_s2e, Wr2_s2e,
                       bl2_s2e.reshape(1, H))

    gs, ge = _make_cls_gather()(s2, e2, ls_r, le_r)
    pred = _cls_dot(gs, ge)
    return pred.reshape(B_PAD)[:B]
